# baseline (speedup 1.0000x reference)
import functools

import jax
import jax.numpy as jnp
from jax import lax
from jax.experimental import pallas as pl
from jax.experimental.pallas import tpu as pltpu
from jax.experimental.pallas import tpu_sc as plsc

N_NODES = 10000
N_EDGES = 160000
N_GRAPHS = 64
N_ATOM = 128
N_BOND = 16
H = 32
N_LAYERS = 2
N_ITERS = 6
N_OUT = 2

E_PAD = 163840
N_PAD = 10240
PK = 4
EROWS = E_PAD // PK
ROWS_W = 40
SUP = 8
NSUP = ROWS_W // SUP
RPS = 640

_SC_PARAMS = pltpu.CompilerParams(use_tc_tiling_on_sc=False)


NROWS = N_PAD // PK


def _node_embed(x_node4, W4n, b4n):
    def body(x_ref, w_ref, b_ref, o_ref):
        o_ref[...] = jnp.dot(x_ref[...], w_ref[...],
                             preferred_element_type=jnp.float32) + b_ref[...]
    blk = 512
    return pl.pallas_call(
        body,
        grid=(NROWS // blk,),
        in_specs=[pl.BlockSpec((blk, PK * N_ATOM), lambda i: (i, 0)),
                  pl.BlockSpec((PK * N_ATOM, PK * H), lambda i: (0, 0)),
                  pl.BlockSpec((1, PK * H), lambda i: (0, 0))],
        out_specs=pl.BlockSpec((blk, PK * H), lambda i: (i, 0)),
        out_shape=jax.ShapeDtypeStruct((NROWS, PK * H), jnp.float32),
    )(x_node4, W4n, b4n)


def _edge_embed(x_edge_raw, W4e, be4, W14, b14):
    def body(x_ref, we_ref, be_ref, w1_ref, b1_ref, o_ref):
        he = jnp.dot(x_ref[...], we_ref[...],
                     preferred_element_type=jnp.float32) + be_ref[...]
        t = jnp.dot(he, w1_ref[...],
                    preferred_element_type=jnp.float32) + b1_ref[...]
        o_ref[...] = jnp.maximum(t, 0.0)
    blk = 2000
    return pl.pallas_call(
        body,
        grid=(N_EDGES // PK // blk,),
        in_specs=[pl.BlockSpec((blk, PK * N_BOND), lambda i: (i, 0)),
                  pl.BlockSpec((PK * N_BOND, PK * H), lambda i: (0, 0)),
                  pl.BlockSpec((1, PK * H), lambda i: (0, 0)),
                  pl.BlockSpec((PK * H, PK * H), lambda i: (0, 0)),
                  pl.BlockSpec((1, PK * H), lambda i: (0, 0))],
        out_specs=pl.BlockSpec((blk, PK * H), lambda i: (i, 0)),
        out_shape=jax.ShapeDtypeStruct((EROWS, PK * H), jnp.float32),
    )(x_edge_raw, W4e, be4, W14, b14)


def _edge_messages(t4, hsrc4, W2rT, Ben2T4):
    RB = 8192

    def body(t_ref, h_ref, w_ref, b_ref, o_ref):
        bf = jnp.bfloat16
        tT = t_ref[...].T.astype(bf)
        hT = h_ref[...].T
        hT16 = hT.astype(bf)
        w16 = w_ref[...].astype(bf)
        parts = []
        for m in range(PK):
            tj = tT[32 * m:32 * m + 32]
            hj = hT16[32 * m:32 * m + 32]
            uT = (tj[:, None, :] * hj[None, :, :]).reshape(H * H, RB)
            parts.append(jnp.dot(w16, uT,
                                 preferred_element_type=jnp.float32))
        msgT = jnp.concatenate(parts, axis=0)
        msgT = msgT + jnp.dot(b_ref[...], hT,
                              preferred_element_type=jnp.float32)
        o_ref[...] = msgT.T

    return pl.pallas_call(
        body,
        grid=(EROWS // RB,),
        in_specs=[pl.BlockSpec((RB, PK * H), lambda i: (i, 0)),
                  pl.BlockSpec((RB, PK * H), lambda i: (i, 0)),
                  pl.BlockSpec((H, H * H), lambda i: (0, 0)),
                  pl.BlockSpec((PK * H, PK * H), lambda i: (0, 0))],
        out_specs=pl.BlockSpec((RB, PK * H), lambda i: (i, 0)),
        out_shape=jax.ShapeDtypeStruct((EROWS, PK * H), jnp.float32),
    )(t4, hsrc4, W2rT, Ben2T4)


def _gru_update(aggp4, degp4, h4, Wihp, Whhp, bihp, bhhp):
    def body(a_ref, d_ref, h_ref, wi_ref, wh_ref, bi_ref, bh_ref, o_ref):
        a = a_ref[...]
        agg = a[0] + a[1]
        d = d_ref[...]
        deg = jnp.maximum(d[0] + d[1], 1.0)
        m = jnp.maximum(agg / deg, 0.0)
        hg = h_ref[...]
        gi = jnp.dot(m, wi_ref[...],
                     preferred_element_type=jnp.float32) + bi_ref[...]
        gh = jnp.dot(hg, wh_ref[...],
                     preferred_element_type=jnp.float32) + bh_ref[...]
        G = PK * H
        r = jax.nn.sigmoid(gi[:, 0:G] + gh[:, 0:G])
        zg = jax.nn.sigmoid(gi[:, G:2 * G] + gh[:, G:2 * G])
        n = jnp.tanh(gi[:, 2 * G:3 * G] + r * gh[:, 2 * G:3 * G])
        o_ref[...] = (1.0 - zg) * n + zg * hg
    blk = 512
    return pl.pallas_call(
        body,
        grid=(NROWS // blk,),
        in_specs=[pl.BlockSpec((2, blk, PK * H), lambda i: (0, i, 0)),
                  pl.BlockSpec((2, blk, PK * H), lambda i: (0, i, 0)),
                  pl.BlockSpec((blk, PK * H), lambda i: (i, 0)),
                  pl.BlockSpec((PK * H, 3 * PK * H), lambda i: (0, 0)),
                  pl.BlockSpec((PK * H, 3 * PK * H), lambda i: (0, 0)),
                  pl.BlockSpec((1, 3 * PK * H), lambda i: (0, 0)),
                  pl.BlockSpec((1, 3 * PK * H), lambda i: (0, 0))],
        out_specs=pl.BlockSpec((blk, PK * H), lambda i: (i, 0)),
        out_shape=jax.ShapeDtypeStruct((NROWS, PK * H), jnp.float32),
    )(aggp4, degp4, h4, Wihp, Whhp, bihp, bhhp)


def _set2set(h, gid2d, Wih0, Whh0, bih0, bhh0, Wih1, Whh1, bih1, bhh1,
             Wc1, bc1, Wc2, bc2):
    def lstm(x, hs, cs, Wih, Whh, bih, bhh):
        g = (jnp.dot(x, Wih, preferred_element_type=jnp.float32) + bih
             + jnp.dot(hs, Whh, preferred_element_type=jnp.float32) + bhh)
        i = jax.nn.sigmoid(g[:, 0:H])
        f = jax.nn.sigmoid(g[:, H:2 * H])
        gg = jnp.tanh(g[:, 2 * H:3 * H])
        o = jax.nn.sigmoid(g[:, 3 * H:4 * H])
        c_new = f * cs + i * gg
        return o * jnp.tanh(c_new), c_new

    def body(h_ref, gid_ref, wi0, wh0, bi0, bh0, wi1, wh1, bi1, bh1,
             wc1, bc1_, wc2, bc2_, o_ref):
        hh = h_ref[...]
        hT = hh.T
        gid = gid_ref[...]
        ohT = (gid == lax.broadcasted_iota(jnp.int32, (N_GRAPHS, N_NODES), 0)
               ).astype(jnp.float32)
        q_star = jnp.zeros((N_GRAPHS, 2 * H), jnp.float32)
        h0 = jnp.zeros((N_GRAPHS, H), jnp.float32)
        c0 = jnp.zeros((N_GRAPHS, H), jnp.float32)
        h1 = jnp.zeros((N_GRAPHS, H), jnp.float32)
        c1 = jnp.zeros((N_GRAPHS, H), jnp.float32)
        for _ in range(N_ITERS):
            h0, c0 = lstm(q_star, h0, c0, wi0[...], wh0[...], bi0[...],
                          bh0[...])
            h1, c1 = lstm(h0, h1, c1, wi1[...], wh1[...], bi1[...],
                          bh1[...])
            q = h1
            scoresT = jnp.dot(q, hT,
                              preferred_element_type=jnp.float32)
            eT = jnp.sum(ohT * scoresT, axis=0, keepdims=True)
            maskedT = jnp.where(ohT > 0.0, eT, -1e30)
            emax = jnp.max(maskedT, axis=1, keepdims=True)
            emax_n = jnp.sum(ohT * emax, axis=0, keepdims=True)
            e_expT = jnp.exp(eT - emax_n)
            esum = jnp.sum(ohT * e_expT, axis=1, keepdims=True)
            esum_n = jnp.sum(ohT * esum, axis=0, keepdims=True)
            alphaT = e_expT / esum_n
            w = ohT * alphaT
            readout = jnp.dot(w, hh,
                              preferred_element_type=jnp.float32)
            q_star = jnp.concatenate([q, readout], axis=1)
        z = jnp.maximum(
            jnp.dot(q_star, wc1[...],
                    preferred_element_type=jnp.float32) + bc1_[...], 0.0)
        o_ref[...] = jnp.dot(z, wc2[...],
                             preferred_element_type=jnp.float32) + bc2_[...]

    full = lambda shape: pl.BlockSpec(shape, lambda: tuple(0 for _ in shape))
    return pl.pallas_call(
        body,
        in_specs=[full((N_NODES, H)), full((1, N_NODES)),
                  full((2 * H, 4 * H)), full((H, 4 * H)),
                  full((1, 4 * H)), full((1, 4 * H)),
                  full((H, 4 * H)), full((H, 4 * H)),
                  full((1, 4 * H)), full((1, 4 * H)),
                  full((2 * H, H)), full((1, H)),
                  full((H, N_OUT)), full((1, N_OUT))],
        out_specs=full((N_GRAPHS, N_OUT)),
        out_shape=jax.ShapeDtypeStruct((N_GRAPHS, N_OUT), jnp.float32),
    )(h, gid2d, Wih0, Whh0, bih0, bhh0, Wih1, Whh1, bih1, bhh1,
      Wc1, bc1, Wc2, bc2)


def _sc_gather(table, idx2d):
    mesh = plsc.VectorSubcoreMesh(core_axis_name="c", subcore_axis_name="s")

    @functools.partial(
        pl.kernel, mesh=mesh, compiler_params=_SC_PARAMS,
        out_type=jax.ShapeDtypeStruct((E_PAD, H), jnp.float32),
        scratch_types=[pltpu.VMEM((ROWS_W, 128), jnp.int32),
                       pltpu.VMEM((2, SUP * 128, H), jnp.float32),
                       pltpu.VMEM_SHARED((N_PAD, H), jnp.float32),
                       pltpu.SemaphoreType.DMA],
    )
    def k(table_h, idx_h, out_h, idx_v, rows_v, table_s, sem):
        cid = lax.axis_index("c")
        sid = lax.axis_index("s")
        wid = sid * 2 + cid
        row0 = wid * ROWS_W

        @pl.when(sid == 0)
        def _():
            pltpu.sync_copy(table_h, table_s)
        plsc.subcore_barrier()

        pltpu.sync_copy(idx_h.at[pl.ds(row0, ROWS_W)], idx_v)
        ebase = row0 * 128

        def fire(s, b):
            return [pltpu.async_copy(table_s.at[idx_v.at[s * SUP + j]],
                                     rows_v.at[b].at[pl.ds(j * 128, 128)],
                                     sem)
                    for j in range(SUP)]

        ga = {0: fire(0, 0)}
        for s in range(NSUP):
            b = s % 2
            if s + 1 < NSUP:
                ga[(s + 1) % 2] = fire(s + 1, (s + 1) % 2)
            for hd in ga[b]:
                hd.wait()
            pltpu.sync_copy(rows_v.at[b],
                            out_h.at[pl.ds(ebase + s * SUP * 128,
                                           SUP * 128)])

    return k(table, idx2d).reshape(EROWS, PK * H)


def _sc_scatter_add(msg4, dst2d, zero_init):
    mesh = plsc.VectorSubcoreMesh(core_axis_name="c", subcore_axis_name="s")

    @functools.partial(
        pl.kernel, mesh=mesh, compiler_params=_SC_PARAMS,
        out_type=jax.ShapeDtypeStruct((2, N_PAD, H), jnp.float32),
        scratch_types=[pltpu.VMEM((ROWS_W, 128), jnp.int32),
                       pltpu.VMEM((2, SUP * 128, H), jnp.float32),
                       pltpu.VMEM_SHARED((N_PAD, H), jnp.float32),
                       pltpu.SemaphoreType.DMA,
                       pltpu.SemaphoreType.DMA],
    )
    def k(msg_h, idx_h, zro_h, out_h, idx_v, msg_v, shared, sem_ld, sem_sc):
        cid = lax.axis_index("c")
        sid = lax.axis_index("s")
        wid = sid * 2 + cid
        row0 = wid * ROWS_W
        pltpu.sync_copy(zro_h.at[pl.ds(sid * RPS, RPS)],
                        shared.at[pl.ds(sid * RPS, RPS)])
        plsc.subcore_barrier()
        pltpu.sync_copy(idx_h.at[pl.ds(row0, ROWS_W)], idx_v)
        ebase = row0 * 128

        def load(s, b):
            return pltpu.async_copy(
                msg_h.at[pl.ds(ebase + s * SUP * 128, SUP * 128)],
                msg_v.at[b], sem_ld)

        ld = {0: load(0, 0)}
        sc = {0: [], 1: []}
        for s in range(NSUP):
            b = s % 2
            nb = (s + 1) % 2
            if s + 1 < NSUP:
                for hd in sc[nb]:
                    hd.wait()
                sc[nb] = []
                ld[nb] = load(s + 1, nb)
            ld[b].wait()
            sc[b] = [pltpu.async_copy(msg_v.at[b].at[pl.ds(j * 128, 128)],
                                      shared.at[idx_v.at[s * SUP + j]],
                                      sem_sc, add=True)
                     for j in range(SUP)]
        for bb in (0, 1):
            for hd in sc[bb]:
                hd.wait()
        plsc.subcore_barrier()
        pltpu.sync_copy(shared.at[pl.ds(sid * RPS, RPS)],
                        out_h.at[cid, pl.ds(sid * RPS, RPS)])

    return k(msg4.reshape(E_PAD, H), dst2d, zero_init)


def _sc_degree(dst2d, zero_init, ones_rows):
    mesh = plsc.VectorSubcoreMesh(core_axis_name="c", subcore_axis_name="s")

    @functools.partial(
        pl.kernel, mesh=mesh, compiler_params=_SC_PARAMS,
        out_type=jax.ShapeDtypeStruct((2, N_PAD, H), jnp.float32),
        scratch_types=[pltpu.VMEM((ROWS_W, 128), jnp.int32),
                       pltpu.VMEM((128, H), jnp.float32),
                       pltpu.VMEM_SHARED((N_PAD, H), jnp.float32),
                       pltpu.SemaphoreType.DMA],
    )
    def k(idx_h, zro_h, ones_h, out_h, idx_v, ones_v, shared, sem):
        cid = lax.axis_index("c")
        sid = lax.axis_index("s")
        wid = sid * 2 + cid
        row0 = wid * ROWS_W
        pltpu.sync_copy(zro_h.at[pl.ds(sid * RPS, RPS)],
                        shared.at[pl.ds(sid * RPS, RPS)])
        plsc.subcore_barrier()
        pltpu.sync_copy(ones_h, ones_v)
        pltpu.sync_copy(idx_h.at[pl.ds(row0, ROWS_W)], idx_v)
        sc = []
        for r in range(ROWS_W):
            sc.append(pltpu.async_copy(ones_v, shared.at[idx_v.at[r]],
                                       sem, add=True))
        for hd in sc:
            hd.wait()
        plsc.subcore_barrier()
        pltpu.sync_copy(shared.at[pl.ds(sid * RPS, RPS)],
                        out_h.at[cid, pl.ds(sid * RPS, RPS)])

    return k(dst2d, zero_init, ones_rows)


def kernel(x_node, x_edge, edge_index, node_graph_ids,
           Wn, bn, We, be, Wen1, ben1, Wen2, ben2,
           gru_Wih, gru_Whh, gru_bih, gru_bhh,
           lstm_Wih0, lstm_Whh0, lstm_bih0, lstm_bhh0,
           lstm_Wih1, lstm_Whh1, lstm_bih1, lstm_bhh1,
           Wc1, bc1, Wc2, bc2):
    f32 = jnp.float32
    src = edge_index[0]
    dst = edge_index[1]
    pad = E_PAD - N_EDGES
    src2d = jnp.concatenate(
        [src, jnp.zeros((pad,), jnp.int32)]).reshape(E_PAD // 128, 128)
    dst2d = jnp.concatenate(
        [dst, jnp.full((pad,), N_NODES, jnp.int32)]).reshape(E_PAD // 128, 128)
    xe4 = x_edge.reshape(N_EDGES // PK, PK * N_BOND)
    xn4 = jnp.concatenate(
        [x_node.reshape(N_NODES // PK, PK * N_ATOM),
         jnp.zeros((NROWS - N_NODES // PK, PK * N_ATOM), f32)])
    eyePK = jnp.eye(PK, dtype=f32)
    W4n = jnp.kron(eyePK, Wn)
    b4n = jnp.tile(bn, PK).reshape(1, PK * H)
    W4e = jnp.kron(eyePK, We)
    W14 = jnp.kron(eyePK, Wen1)
    be4 = jnp.tile(be, PK).reshape(1, PK * H)
    b14 = jnp.tile(ben1, PK).reshape(1, PK * H)
    W2rT = Wen2.reshape(H, H, H).transpose(2, 0, 1).reshape(H, H * H)
    Ben2T4 = jnp.kron(eyePK, ben2.reshape(H, H).T)
    def gru_pack_w(W):
        return (jnp.kron(eyePK, W).reshape(PK * H, PK, 3, H)
                .transpose(0, 2, 1, 3).reshape(PK * H, 3 * PK * H))

    def gru_pack_b(b):
        return jnp.tile(b.reshape(3, H), (1, PK)).reshape(1, 3 * PK * H)
    Wihp = gru_pack_w(gru_Wih)
    Whhp = gru_pack_w(gru_Whh)
    bihp = gru_pack_b(gru_bih)
    bhhp = gru_pack_b(gru_bhh)
    zeroN = jnp.zeros((N_PAD, H), f32)
    ones_rows = jnp.ones((128, H), f32)
    gid2d = node_graph_ids.reshape(1, N_NODES)

    h4 = _node_embed(xn4, W4n, b4n)
    t4 = _edge_embed(xe4, W4e, be4, W14, b14)
    degp4 = _sc_degree(dst2d, zeroN, ones_rows).reshape(2, NROWS, PK * H)

    for _ in range(N_LAYERS):
        hsrc4 = _sc_gather(h4.reshape(N_PAD, H), src2d)
        msg4 = _edge_messages(t4, hsrc4, W2rT, Ben2T4)
        aggp4 = _sc_scatter_add(msg4, dst2d, zeroN).reshape(2, NROWS, PK * H)
        h4 = _gru_update(aggp4, degp4, h4, Wihp, Whhp, bihp, bhhp)

    h_fin = h4.reshape(N_PAD, H)[:N_NODES]
    out = _set2set(h_fin, gid2d,
                   lstm_Wih0, lstm_Whh0, lstm_bih0.reshape(1, 4 * H),
                   lstm_bhh0.reshape(1, 4 * H),
                   lstm_Wih1, lstm_Whh1, lstm_bih1.reshape(1, 4 * H),
                   lstm_bhh1.reshape(1, 4 * H),
                   Wc1, bc1.reshape(1, H), Wc2, bc2.reshape(1, N_OUT))
    return out


# speedup vs baseline: 1.1040x; 1.1040x over previous
import functools

import jax
import jax.numpy as jnp
from jax import lax
from jax.experimental import pallas as pl
from jax.experimental.pallas import tpu as pltpu
from jax.experimental.pallas import tpu_sc as plsc

N_NODES = 10000
N_EDGES = 160000
N_GRAPHS = 64
N_ATOM = 128
N_BOND = 16
H = 32
N_LAYERS = 2
N_ITERS = 6
N_OUT = 2

E_PAD = 163840
N_PAD = 10240
PK = 4
EROWS = E_PAD // PK
ROWS_W = 40
SUP = 8
NSUP = ROWS_W // SUP
RPS = 640

_SC_PARAMS = pltpu.CompilerParams(use_tc_tiling_on_sc=False)


NROWS = N_PAD // PK


def _node_embed(x_node4, W4n, b4n):
    def body(x_ref, w_ref, b_ref, o_ref):
        o_ref[...] = jnp.dot(x_ref[...], w_ref[...],
                             preferred_element_type=jnp.float32) + b_ref[...]
    blk = 512
    return pl.pallas_call(
        body,
        grid=(NROWS // blk,),
        in_specs=[pl.BlockSpec((blk, PK * N_ATOM), lambda i: (i, 0)),
                  pl.BlockSpec((PK * N_ATOM, PK * H), lambda i: (0, 0)),
                  pl.BlockSpec((1, PK * H), lambda i: (0, 0))],
        out_specs=pl.BlockSpec((blk, PK * H), lambda i: (i, 0)),
        out_shape=jax.ShapeDtypeStruct((NROWS, PK * H), jnp.float32),
    )(x_node4, W4n, b4n)


def _edge_embed(x_edge_raw, W4e, be4, W14, b14):
    def body(x_ref, we_ref, be_ref, w1_ref, b1_ref, o_ref):
        he = jnp.dot(x_ref[...], we_ref[...],
                     preferred_element_type=jnp.float32) + be_ref[...]
        t = jnp.dot(he, w1_ref[...],
                    preferred_element_type=jnp.float32) + b1_ref[...]
        o_ref[...] = jnp.maximum(t, 0.0)
    blk = 2000
    return pl.pallas_call(
        body,
        grid=(N_EDGES // PK // blk,),
        in_specs=[pl.BlockSpec((blk, PK * N_BOND), lambda i: (i, 0)),
                  pl.BlockSpec((PK * N_BOND, PK * H), lambda i: (0, 0)),
                  pl.BlockSpec((1, PK * H), lambda i: (0, 0)),
                  pl.BlockSpec((PK * H, PK * H), lambda i: (0, 0)),
                  pl.BlockSpec((1, PK * H), lambda i: (0, 0))],
        out_specs=pl.BlockSpec((blk, PK * H), lambda i: (i, 0)),
        out_shape=jax.ShapeDtypeStruct((EROWS, PK * H), jnp.float32),
    )(x_edge_raw, W4e, be4, W14, b14)


def _edge_messages_fused(xe4, hsrc4, W4e, be4, W14, b14, W2rT, Ben2T4):
    RB = 4000

    def body(x_ref, h_ref, we_ref, be_ref, w1_ref, b1_ref, w_ref, b_ref,
             o_ref, t_ref):
        bf = jnp.bfloat16
        he = jnp.dot(x_ref[...], we_ref[...],
                     preferred_element_type=jnp.float32) + be_ref[...]
        t_blk = jnp.maximum(
            jnp.dot(he, w1_ref[...],
                    preferred_element_type=jnp.float32) + b1_ref[...], 0.0)
        t_ref[...] = t_blk
        tT = t_blk.T.astype(bf)
        hT = h_ref[...].T
        hT16 = hT.astype(bf)
        w16 = w_ref[...].astype(bf)
        parts = []
        for m in range(PK):
            tj = tT[32 * m:32 * m + 32]
            hj = hT16[32 * m:32 * m + 32]
            uT = (tj[:, None, :] * hj[None, :, :]).reshape(H * H, RB)
            parts.append(jnp.dot(w16, uT,
                                 preferred_element_type=jnp.float32))
        msgT = jnp.concatenate(parts, axis=0)
        msgT = msgT + jnp.dot(b_ref[...], hT,
                              preferred_element_type=jnp.float32)
        o_ref[...] = msgT.T

    return pl.pallas_call(
        body,
        grid=(N_EDGES // PK // RB,),
        in_specs=[pl.BlockSpec((RB, PK * N_BOND), lambda i: (i, 0)),
                  pl.BlockSpec((RB, PK * H), lambda i: (i, 0)),
                  pl.BlockSpec((PK * N_BOND, PK * H), lambda i: (0, 0)),
                  pl.BlockSpec((1, PK * H), lambda i: (0, 0)),
                  pl.BlockSpec((PK * H, PK * H), lambda i: (0, 0)),
                  pl.BlockSpec((1, PK * H), lambda i: (0, 0)),
                  pl.BlockSpec((H, H * H), lambda i: (0, 0)),
                  pl.BlockSpec((PK * H, PK * H), lambda i: (0, 0))],
        out_specs=[pl.BlockSpec((RB, PK * H), lambda i: (i, 0)),
                   pl.BlockSpec((RB, PK * H), lambda i: (i, 0))],
        out_shape=[jax.ShapeDtypeStruct((EROWS, PK * H), jnp.float32),
                   jax.ShapeDtypeStruct((EROWS, PK * H), jnp.float32)],
    )(xe4, hsrc4, W4e, be4, W14, b14, W2rT, Ben2T4)


def _edge_messages(t4, hsrc4, W2rT, Ben2T4):
    RB = 4096

    def body(t_ref, h_ref, w_ref, b_ref, o_ref):
        bf = jnp.bfloat16
        tT = t_ref[...].T.astype(bf)
        hT = h_ref[...].T
        hT16 = hT.astype(bf)
        w16 = w_ref[...].astype(bf)
        parts = []
        for m in range(PK):
            tj = tT[32 * m:32 * m + 32]
            hj = hT16[32 * m:32 * m + 32]
            uT = (tj[:, None, :] * hj[None, :, :]).reshape(H * H, RB)
            parts.append(jnp.dot(w16, uT,
                                 preferred_element_type=jnp.float32))
        msgT = jnp.concatenate(parts, axis=0)
        msgT = msgT + jnp.dot(b_ref[...], hT,
                              preferred_element_type=jnp.float32)
        o_ref[...] = msgT.T

    return pl.pallas_call(
        body,
        grid=(EROWS // RB,),
        in_specs=[pl.BlockSpec((RB, PK * H), lambda i: (i, 0)),
                  pl.BlockSpec((RB, PK * H), lambda i: (i, 0)),
                  pl.BlockSpec((H, H * H), lambda i: (0, 0)),
                  pl.BlockSpec((PK * H, PK * H), lambda i: (0, 0))],
        out_specs=pl.BlockSpec((RB, PK * H), lambda i: (i, 0)),
        out_shape=jax.ShapeDtypeStruct((EROWS, PK * H), jnp.float32),
    )(t4, hsrc4, W2rT, Ben2T4)


def _gru_update(aggp4, degp4, h4, Wihp, Whhp, bihp, bhhp):
    def body(a_ref, d_ref, h_ref, wi_ref, wh_ref, bi_ref, bh_ref, o_ref):
        a = a_ref[...]
        agg = a[0] + a[1]
        d = d_ref[...]
        deg = jnp.maximum(d[0] + d[1], 1.0)
        m = jnp.maximum(agg / deg, 0.0)
        hg = h_ref[...]
        gi = jnp.dot(m, wi_ref[...],
                     preferred_element_type=jnp.float32) + bi_ref[...]
        gh = jnp.dot(hg, wh_ref[...],
                     preferred_element_type=jnp.float32) + bh_ref[...]
        G = PK * H
        r = jax.nn.sigmoid(gi[:, 0:G] + gh[:, 0:G])
        zg = jax.nn.sigmoid(gi[:, G:2 * G] + gh[:, G:2 * G])
        n = jnp.tanh(gi[:, 2 * G:3 * G] + r * gh[:, 2 * G:3 * G])
        o_ref[...] = (1.0 - zg) * n + zg * hg
    blk = 512
    return pl.pallas_call(
        body,
        grid=(NROWS // blk,),
        in_specs=[pl.BlockSpec((2, blk, PK * H), lambda i: (0, i, 0)),
                  pl.BlockSpec((2, blk, PK * H), lambda i: (0, i, 0)),
                  pl.BlockSpec((blk, PK * H), lambda i: (i, 0)),
                  pl.BlockSpec((PK * H, 3 * PK * H), lambda i: (0, 0)),
                  pl.BlockSpec((PK * H, 3 * PK * H), lambda i: (0, 0)),
                  pl.BlockSpec((1, 3 * PK * H), lambda i: (0, 0)),
                  pl.BlockSpec((1, 3 * PK * H), lambda i: (0, 0))],
        out_specs=pl.BlockSpec((blk, PK * H), lambda i: (i, 0)),
        out_shape=jax.ShapeDtypeStruct((NROWS, PK * H), jnp.float32),
    )(aggp4, degp4, h4, Wihp, Whhp, bihp, bhhp)


def _set2set(h, gid2d, Wih0, Whh0, bih0, bhh0, Wih1, Whh1, bih1, bhh1,
             Wc1, bc1, Wc2, bc2):
    def lstm(x, hs, cs, Wih, Whh, bih, bhh):
        g = (jnp.dot(x, Wih, preferred_element_type=jnp.float32) + bih
             + jnp.dot(hs, Whh, preferred_element_type=jnp.float32) + bhh)
        i = jax.nn.sigmoid(g[:, 0:H])
        f = jax.nn.sigmoid(g[:, H:2 * H])
        gg = jnp.tanh(g[:, 2 * H:3 * H])
        o = jax.nn.sigmoid(g[:, 3 * H:4 * H])
        c_new = f * cs + i * gg
        return o * jnp.tanh(c_new), c_new

    def body(h_ref, gid_ref, wi0, wh0, bi0, bh0, wi1, wh1, bi1, bh1,
             wc1, bc1_, wc2, bc2_, o_ref):
        hh = h_ref[...]
        hT = hh.T
        gid = gid_ref[...]
        ohT = (gid == lax.broadcasted_iota(jnp.int32, (N_GRAPHS, N_NODES), 0)
               ).astype(jnp.float32)
        q_star = jnp.zeros((N_GRAPHS, 2 * H), jnp.float32)
        h0 = jnp.zeros((N_GRAPHS, H), jnp.float32)
        c0 = jnp.zeros((N_GRAPHS, H), jnp.float32)
        h1 = jnp.zeros((N_GRAPHS, H), jnp.float32)
        c1 = jnp.zeros((N_GRAPHS, H), jnp.float32)
        for _ in range(N_ITERS):
            h0, c0 = lstm(q_star, h0, c0, wi0[...], wh0[...], bi0[...],
                          bh0[...])
            h1, c1 = lstm(h0, h1, c1, wi1[...], wh1[...], bi1[...],
                          bh1[...])
            q = h1
            scoresT = jnp.dot(q, hT,
                              preferred_element_type=jnp.float32)
            eT = jnp.sum(ohT * scoresT, axis=0, keepdims=True)
            maskedT = jnp.where(ohT > 0.0, eT, -1e30)
            emax = jnp.max(maskedT, axis=1, keepdims=True)
            emax_n = jnp.sum(ohT * emax, axis=0, keepdims=True)
            e_expT = jnp.exp(eT - emax_n)
            esum = jnp.sum(ohT * e_expT, axis=1, keepdims=True)
            esum_n = jnp.sum(ohT * esum, axis=0, keepdims=True)
            alphaT = e_expT / esum_n
            w = ohT * alphaT
            readout = jnp.dot(w, hh,
                              preferred_element_type=jnp.float32)
            q_star = jnp.concatenate([q, readout], axis=1)
        z = jnp.maximum(
            jnp.dot(q_star, wc1[...],
                    preferred_element_type=jnp.float32) + bc1_[...], 0.0)
        o_ref[...] = jnp.dot(z, wc2[...],
                             preferred_element_type=jnp.float32) + bc2_[...]

    full = lambda shape: pl.BlockSpec(shape, lambda: tuple(0 for _ in shape))
    return pl.pallas_call(
        body,
        in_specs=[full((N_NODES, H)), full((1, N_NODES)),
                  full((2 * H, 4 * H)), full((H, 4 * H)),
                  full((1, 4 * H)), full((1, 4 * H)),
                  full((H, 4 * H)), full((H, 4 * H)),
                  full((1, 4 * H)), full((1, 4 * H)),
                  full((2 * H, H)), full((1, H)),
                  full((H, N_OUT)), full((1, N_OUT))],
        out_specs=full((N_GRAPHS, N_OUT)),
        out_shape=jax.ShapeDtypeStruct((N_GRAPHS, N_OUT), jnp.float32),
    )(h, gid2d, Wih0, Whh0, bih0, bhh0, Wih1, Whh1, bih1, bhh1,
      Wc1, bc1, Wc2, bc2)


def _sc_gather(table, idx2d):
    mesh = plsc.VectorSubcoreMesh(core_axis_name="c", subcore_axis_name="s")

    @functools.partial(
        pl.kernel, mesh=mesh, compiler_params=_SC_PARAMS,
        out_type=jax.ShapeDtypeStruct((E_PAD, H), jnp.float32),
        scratch_types=[pltpu.VMEM((ROWS_W, 128), jnp.int32),
                       pltpu.VMEM((2, SUP * 128, H), jnp.float32),
                       pltpu.VMEM_SHARED((N_PAD, H), jnp.float32),
                       pltpu.SemaphoreType.DMA],
    )
    def k(table_h, idx_h, out_h, idx_v, rows_v, table_s, sem):
        cid = lax.axis_index("c")
        sid = lax.axis_index("s")
        wid = sid * 2 + cid
        row0 = wid * ROWS_W

        @pl.when(sid == 0)
        def _():
            pltpu.sync_copy(table_h, table_s)
        plsc.subcore_barrier()

        pltpu.sync_copy(idx_h.at[pl.ds(row0, ROWS_W)], idx_v)
        ebase = row0 * 128

        def fire(s, b):
            return [pltpu.async_copy(table_s.at[idx_v.at[s * SUP + j]],
                                     rows_v.at[b].at[pl.ds(j * 128, 128)],
                                     sem)
                    for j in range(SUP)]

        ga = {0: fire(0, 0)}
        for s in range(NSUP):
            b = s % 2
            if s + 1 < NSUP:
                ga[(s + 1) % 2] = fire(s + 1, (s + 1) % 2)
            for hd in ga[b]:
                hd.wait()
            pltpu.sync_copy(rows_v.at[b],
                            out_h.at[pl.ds(ebase + s * SUP * 128,
                                           SUP * 128)])

    return k(table, idx2d).reshape(EROWS, PK * H)


def _sc_scatter_add(msg4, dst2d, zero_init):
    mesh = plsc.VectorSubcoreMesh(core_axis_name="c", subcore_axis_name="s")

    @functools.partial(
        pl.kernel, mesh=mesh, compiler_params=_SC_PARAMS,
        out_type=jax.ShapeDtypeStruct((2, N_PAD, H), jnp.float32),
        scratch_types=[pltpu.VMEM((ROWS_W, 128), jnp.int32),
                       pltpu.VMEM((2, SUP * 128, H), jnp.float32),
                       pltpu.VMEM_SHARED((N_PAD, H), jnp.float32),
                       pltpu.SemaphoreType.DMA,
                       pltpu.SemaphoreType.DMA],
    )
    def k(msg_h, idx_h, zro_h, out_h, idx_v, msg_v, shared, sem_ld, sem_sc):
        cid = lax.axis_index("c")
        sid = lax.axis_index("s")
        wid = sid * 2 + cid
        row0 = wid * ROWS_W
        pltpu.sync_copy(zro_h.at[pl.ds(sid * RPS, RPS)],
                        shared.at[pl.ds(sid * RPS, RPS)])
        plsc.subcore_barrier()
        pltpu.sync_copy(idx_h.at[pl.ds(row0, ROWS_W)], idx_v)
        ebase = row0 * 128

        def load(s, b):
            return pltpu.async_copy(
                msg_h.at[pl.ds(ebase + s * SUP * 128, SUP * 128)],
                msg_v.at[b], sem_ld)

        ld = {0: load(0, 0)}
        sc = {0: [], 1: []}
        for s in range(NSUP):
            b = s % 2
            nb = (s + 1) % 2
            if s + 1 < NSUP:
                for hd in sc[nb]:
                    hd.wait()
                sc[nb] = []
                ld[nb] = load(s + 1, nb)
            ld[b].wait()
            sc[b] = [pltpu.async_copy(msg_v.at[b].at[pl.ds(j * 128, 128)],
                                      shared.at[idx_v.at[s * SUP + j]],
                                      sem_sc, add=True)
                     for j in range(SUP)]
        for bb in (0, 1):
            for hd in sc[bb]:
                hd.wait()
        plsc.subcore_barrier()
        pltpu.sync_copy(shared.at[pl.ds(sid * RPS, RPS)],
                        out_h.at[cid, pl.ds(sid * RPS, RPS)])

    return k(msg4.reshape(E_PAD, H), dst2d, zero_init)


def _sc_degree(dst2d, zero_init, ones_rows):
    mesh = plsc.VectorSubcoreMesh(core_axis_name="c", subcore_axis_name="s")

    @functools.partial(
        pl.kernel, mesh=mesh, compiler_params=_SC_PARAMS,
        out_type=jax.ShapeDtypeStruct((2, N_PAD, H), jnp.float32),
        scratch_types=[pltpu.VMEM((ROWS_W, 128), jnp.int32),
                       pltpu.VMEM((128, H), jnp.float32),
                       pltpu.VMEM_SHARED((N_PAD, H), jnp.float32),
                       pltpu.SemaphoreType.DMA],
    )
    def k(idx_h, zro_h, ones_h, out_h, idx_v, ones_v, shared, sem):
        cid = lax.axis_index("c")
        sid = lax.axis_index("s")
        wid = sid * 2 + cid
        row0 = wid * ROWS_W
        pltpu.sync_copy(zro_h.at[pl.ds(sid * RPS, RPS)],
                        shared.at[pl.ds(sid * RPS, RPS)])
        plsc.subcore_barrier()
        pltpu.sync_copy(ones_h, ones_v)
        pltpu.sync_copy(idx_h.at[pl.ds(row0, ROWS_W)], idx_v)
        sc = []
        for r in range(ROWS_W):
            sc.append(pltpu.async_copy(ones_v, shared.at[idx_v.at[r]],
                                       sem, add=True))
        for hd in sc:
            hd.wait()
        plsc.subcore_barrier()
        pltpu.sync_copy(shared.at[pl.ds(sid * RPS, RPS)],
                        out_h.at[cid, pl.ds(sid * RPS, RPS)])

    return k(dst2d, zero_init, ones_rows)


def kernel(x_node, x_edge, edge_index, node_graph_ids,
           Wn, bn, We, be, Wen1, ben1, Wen2, ben2,
           gru_Wih, gru_Whh, gru_bih, gru_bhh,
           lstm_Wih0, lstm_Whh0, lstm_bih0, lstm_bhh0,
           lstm_Wih1, lstm_Whh1, lstm_bih1, lstm_bhh1,
           Wc1, bc1, Wc2, bc2):
    f32 = jnp.float32
    src = edge_index[0]
    dst = edge_index[1]
    pad = E_PAD - N_EDGES
    src2d = jnp.concatenate(
        [src, jnp.zeros((pad,), jnp.int32)]).reshape(E_PAD // 128, 128)
    dst2d = jnp.concatenate(
        [dst, jnp.full((pad,), N_NODES, jnp.int32)]).reshape(E_PAD // 128, 128)
    xe4 = x_edge.reshape(N_EDGES // PK, PK * N_BOND)
    xn4 = jnp.concatenate(
        [x_node.reshape(N_NODES // PK, PK * N_ATOM),
         jnp.zeros((NROWS - N_NODES // PK, PK * N_ATOM), f32)])
    eyePK = jnp.eye(PK, dtype=f32)
    W4n = jnp.kron(eyePK, Wn)
    b4n = jnp.tile(bn, PK).reshape(1, PK * H)
    W4e = jnp.kron(eyePK, We)
    W14 = jnp.kron(eyePK, Wen1)
    be4 = jnp.tile(be, PK).reshape(1, PK * H)
    b14 = jnp.tile(ben1, PK).reshape(1, PK * H)
    W2rT = Wen2.reshape(H, H, H).transpose(2, 0, 1).reshape(H, H * H)
    Ben2T4 = jnp.kron(eyePK, ben2.reshape(H, H).T)
    def gru_pack_w(W):
        return (jnp.kron(eyePK, W).reshape(PK * H, PK, 3, H)
                .transpose(0, 2, 1, 3).reshape(PK * H, 3 * PK * H))

    def gru_pack_b(b):
        return jnp.tile(b.reshape(3, H), (1, PK)).reshape(1, 3 * PK * H)
    Wihp = gru_pack_w(gru_Wih)
    Whhp = gru_pack_w(gru_Whh)
    bihp = gru_pack_b(gru_bih)
    bhhp = gru_pack_b(gru_bhh)
    zeroN = jnp.zeros((N_PAD, H), f32)
    ones_rows = jnp.ones((128, H), f32)
    gid2d = node_graph_ids.reshape(1, N_NODES)

    h4 = _node_embed(xn4, W4n, b4n)
    degp4 = _sc_degree(dst2d, zeroN, ones_rows).reshape(2, NROWS, PK * H)

    hsrc4 = _sc_gather(h4.reshape(N_PAD, H), src2d)
    msg4, t4 = _edge_messages_fused(xe4, hsrc4, W4e, be4, W14, b14,
                                    W2rT, Ben2T4)
    aggp4 = _sc_scatter_add(msg4, dst2d, zeroN).reshape(2, NROWS, PK * H)
    h4 = _gru_update(aggp4, degp4, h4, Wihp, Whhp, bihp, bhhp)
    hsrc4 = _sc_gather(h4.reshape(N_PAD, H), src2d)
    msg4 = _edge_messages(t4, hsrc4, W2rT, Ben2T4)
    aggp4 = _sc_scatter_add(msg4, dst2d, zeroN).reshape(2, NROWS, PK * H)
    h4 = _gru_update(aggp4, degp4, h4, Wihp, Whhp, bihp, bhhp)

    h_fin = h4.reshape(N_PAD, H)[:N_NODES]
    out = _set2set(h_fin, gid2d,
                   lstm_Wih0, lstm_Whh0, lstm_bih0.reshape(1, 4 * H),
                   lstm_bhh0.reshape(1, 4 * H),
                   lstm_Wih1, lstm_Whh1, lstm_bih1.reshape(1, 4 * H),
                   lstm_bhh1.reshape(1, 4 * H),
                   Wc1, bc1.reshape(1, H), Wc2, bc2.reshape(1, N_OUT))
    return out


# speedup vs baseline: 1.1122x; 1.0074x over previous
import functools

import jax
import jax.numpy as jnp
from jax import lax
from jax.experimental import pallas as pl
from jax.experimental.pallas import tpu as pltpu
from jax.experimental.pallas import tpu_sc as plsc

N_NODES = 10000
N_EDGES = 160000
N_GRAPHS = 64
N_ATOM = 128
N_BOND = 16
H = 32
N_LAYERS = 2
N_ITERS = 6
N_OUT = 2

E_PAD = 163840
N_PAD = 10240
PK = 4
EROWS = E_PAD // PK
ROWS_W = 40
SUP = 8
NSUP = ROWS_W // SUP
RPS = 640

_SC_PARAMS = pltpu.CompilerParams(use_tc_tiling_on_sc=False)


NROWS = N_PAD // PK


def _node_embed(x_node4, W4n, b4n):
    def body(x_ref, w_ref, b_ref, o_ref):
        o_ref[...] = jnp.dot(x_ref[...], w_ref[...],
                             preferred_element_type=jnp.float32) + b_ref[...]
    blk = 512
    return pl.pallas_call(
        body,
        grid=(NROWS // blk,),
        in_specs=[pl.BlockSpec((blk, PK * N_ATOM), lambda i: (i, 0)),
                  pl.BlockSpec((PK * N_ATOM, PK * H), lambda i: (0, 0)),
                  pl.BlockSpec((1, PK * H), lambda i: (0, 0))],
        out_specs=pl.BlockSpec((blk, PK * H), lambda i: (i, 0)),
        out_shape=jax.ShapeDtypeStruct((NROWS, PK * H), jnp.float32),
    )(x_node4, W4n, b4n)


def _edge_embed(x_edge_raw, W4e, be4, W14, b14):
    def body(x_ref, we_ref, be_ref, w1_ref, b1_ref, o_ref):
        he = jnp.dot(x_ref[...], we_ref[...],
                     preferred_element_type=jnp.float32) + be_ref[...]
        t = jnp.dot(he, w1_ref[...],
                    preferred_element_type=jnp.float32) + b1_ref[...]
        o_ref[...] = jnp.maximum(t, 0.0)
    blk = 2000
    return pl.pallas_call(
        body,
        grid=(N_EDGES // PK // blk,),
        in_specs=[pl.BlockSpec((blk, PK * N_BOND), lambda i: (i, 0)),
                  pl.BlockSpec((PK * N_BOND, PK * H), lambda i: (0, 0)),
                  pl.BlockSpec((1, PK * H), lambda i: (0, 0)),
                  pl.BlockSpec((PK * H, PK * H), lambda i: (0, 0)),
                  pl.BlockSpec((1, PK * H), lambda i: (0, 0))],
        out_specs=pl.BlockSpec((blk, PK * H), lambda i: (i, 0)),
        out_shape=jax.ShapeDtypeStruct((EROWS, PK * H), jnp.float32),
    )(x_edge_raw, W4e, be4, W14, b14)


def _edge_messages_fused(xe4, hsrc4, W4e, be4, W14, b14, W2rT, Ben2T4):
    RB = 4000

    def body(x_ref, h_ref, we_ref, be_ref, w1_ref, b1_ref, w_ref, b_ref,
             o_ref, t_ref):
        bf = jnp.bfloat16
        he = jnp.dot(x_ref[...], we_ref[...],
                     preferred_element_type=jnp.float32) + be_ref[...]
        t_blk = jnp.maximum(
            jnp.dot(he, w1_ref[...],
                    preferred_element_type=jnp.float32) + b1_ref[...], 0.0)
        t_ref[...] = t_blk
        tT = t_blk.T.astype(bf)
        hT = h_ref[...].T
        hT16 = hT.astype(bf)
        w16 = w_ref[...].astype(bf)
        parts = []
        for m in range(PK):
            tj = tT[32 * m:32 * m + 32]
            hj = hT16[32 * m:32 * m + 32]
            uT = (tj[:, None, :] * hj[None, :, :]).reshape(H * H, RB)
            parts.append(jnp.dot(w16, uT,
                                 preferred_element_type=jnp.float32))
        msgT = jnp.concatenate(parts, axis=0)
        msgT = msgT + jnp.dot(b_ref[...], hT,
                              preferred_element_type=jnp.float32)
        o_ref[...] = msgT.T

    return pl.pallas_call(
        body,
        grid=(N_EDGES // PK // RB,),
        in_specs=[pl.BlockSpec((RB, PK * N_BOND), lambda i: (i, 0)),
                  pl.BlockSpec((RB, PK * H), lambda i: (i, 0)),
                  pl.BlockSpec((PK * N_BOND, PK * H), lambda i: (0, 0)),
                  pl.BlockSpec((1, PK * H), lambda i: (0, 0)),
                  pl.BlockSpec((PK * H, PK * H), lambda i: (0, 0)),
                  pl.BlockSpec((1, PK * H), lambda i: (0, 0)),
                  pl.BlockSpec((H, H * H), lambda i: (0, 0)),
                  pl.BlockSpec((PK * H, PK * H), lambda i: (0, 0))],
        out_specs=[pl.BlockSpec((RB, PK * H), lambda i: (i, 0)),
                   pl.BlockSpec((RB, PK * H), lambda i: (i, 0))],
        out_shape=[jax.ShapeDtypeStruct((EROWS, PK * H), jnp.float32),
                   jax.ShapeDtypeStruct((EROWS, PK * H), jnp.float32)],
    )(xe4, hsrc4, W4e, be4, W14, b14, W2rT, Ben2T4)


def _edge_messages(t4, hsrc4, W2rT, Ben2T4):
    RB = 4096

    def body(t_ref, h_ref, w_ref, b_ref, o_ref):
        bf = jnp.bfloat16
        tT = t_ref[...].T.astype(bf)
        hT = h_ref[...].T
        hT16 = hT.astype(bf)
        w16 = w_ref[...].astype(bf)
        parts = []
        for m in range(PK):
            tj = tT[32 * m:32 * m + 32]
            hj = hT16[32 * m:32 * m + 32]
            uT = (tj[:, None, :] * hj[None, :, :]).reshape(H * H, RB)
            parts.append(jnp.dot(w16, uT,
                                 preferred_element_type=jnp.float32))
        msgT = jnp.concatenate(parts, axis=0)
        msgT = msgT + jnp.dot(b_ref[...], hT,
                              preferred_element_type=jnp.float32)
        o_ref[...] = msgT.T

    return pl.pallas_call(
        body,
        grid=(EROWS // RB,),
        in_specs=[pl.BlockSpec((RB, PK * H), lambda i: (i, 0)),
                  pl.BlockSpec((RB, PK * H), lambda i: (i, 0)),
                  pl.BlockSpec((H, H * H), lambda i: (0, 0)),
                  pl.BlockSpec((PK * H, PK * H), lambda i: (0, 0))],
        out_specs=pl.BlockSpec((RB, PK * H), lambda i: (i, 0)),
        out_shape=jax.ShapeDtypeStruct((EROWS, PK * H), jnp.float32),
    )(t4, hsrc4, W2rT, Ben2T4)


def _gru_update(aggp4, degp4, h4, Wihp, Whhp, bihp, bhhp):
    def body(a_ref, d_ref, h_ref, wi_ref, wh_ref, bi_ref, bh_ref, o_ref):
        a = a_ref[...]
        agg = a[0] + a[1]
        d = d_ref[...]
        deg = jnp.maximum(d[0] + d[1], 1.0)
        m = jnp.maximum(agg / deg, 0.0)
        hg = h_ref[...]
        gi = jnp.dot(m, wi_ref[...],
                     preferred_element_type=jnp.float32) + bi_ref[...]
        gh = jnp.dot(hg, wh_ref[...],
                     preferred_element_type=jnp.float32) + bh_ref[...]
        G = PK * H
        r = jax.nn.sigmoid(gi[:, 0:G] + gh[:, 0:G])
        zg = jax.nn.sigmoid(gi[:, G:2 * G] + gh[:, G:2 * G])
        n = jnp.tanh(gi[:, 2 * G:3 * G] + r * gh[:, 2 * G:3 * G])
        o_ref[...] = (1.0 - zg) * n + zg * hg
    blk = 512
    return pl.pallas_call(
        body,
        grid=(NROWS // blk,),
        in_specs=[pl.BlockSpec((2, blk, PK * H), lambda i: (0, i, 0)),
                  pl.BlockSpec((2, blk, PK * H), lambda i: (0, i, 0)),
                  pl.BlockSpec((blk, PK * H), lambda i: (i, 0)),
                  pl.BlockSpec((PK * H, 3 * PK * H), lambda i: (0, 0)),
                  pl.BlockSpec((PK * H, 3 * PK * H), lambda i: (0, 0)),
                  pl.BlockSpec((1, 3 * PK * H), lambda i: (0, 0)),
                  pl.BlockSpec((1, 3 * PK * H), lambda i: (0, 0))],
        out_specs=pl.BlockSpec((blk, PK * H), lambda i: (i, 0)),
        out_shape=jax.ShapeDtypeStruct((NROWS, PK * H), jnp.float32),
    )(aggp4, degp4, h4, Wihp, Whhp, bihp, bhhp)


def _set2set(h, gid2d, Wih0, Whh0, bih0, bhh0, Wih1, Whh1, bih1, bhh1,
             Wc1, bc1, Wc2, bc2):
    def lstm(x, hs, cs, Wih, Whh, bih, bhh):
        g = (jnp.dot(x, Wih, preferred_element_type=jnp.float32) + bih
             + jnp.dot(hs, Whh, preferred_element_type=jnp.float32) + bhh)
        i = jax.nn.sigmoid(g[:, 0:H])
        f = jax.nn.sigmoid(g[:, H:2 * H])
        gg = jnp.tanh(g[:, 2 * H:3 * H])
        o = jax.nn.sigmoid(g[:, 3 * H:4 * H])
        c_new = f * cs + i * gg
        return o * jnp.tanh(c_new), c_new

    def body(h_ref, gid_ref, wi0, wh0, bi0, bh0, wi1, wh1, bi1, bh1,
             wc1, bc1_, wc2, bc2_, o_ref):
        hh = h_ref[...]
        hT = hh.T
        gid = gid_ref[...]
        ohT = (gid == lax.broadcasted_iota(jnp.int32, (N_GRAPHS, N_NODES), 0)
               ).astype(jnp.float32)
        q_star = jnp.zeros((N_GRAPHS, 2 * H), jnp.float32)
        h0 = jnp.zeros((N_GRAPHS, H), jnp.float32)
        c0 = jnp.zeros((N_GRAPHS, H), jnp.float32)
        h1 = jnp.zeros((N_GRAPHS, H), jnp.float32)
        c1 = jnp.zeros((N_GRAPHS, H), jnp.float32)
        for _ in range(N_ITERS):
            h0, c0 = lstm(q_star, h0, c0, wi0[...], wh0[...], bi0[...],
                          bh0[...])
            h1, c1 = lstm(h0, h1, c1, wi1[...], wh1[...], bi1[...],
                          bh1[...])
            q = h1
            scoresT = jnp.dot(q, hT,
                              preferred_element_type=jnp.float32)
            eT = jnp.sum(ohT * scoresT, axis=0, keepdims=True)
            maskedT = jnp.where(ohT > 0.0, eT, -1e30)
            emax = jnp.max(maskedT, axis=1, keepdims=True)
            emax_n = jnp.sum(ohT * emax, axis=0, keepdims=True)
            e_expT = jnp.exp(eT - emax_n)
            esum = jnp.sum(ohT * e_expT, axis=1, keepdims=True)
            esum_n = jnp.sum(ohT * esum, axis=0, keepdims=True)
            alphaT = e_expT / esum_n
            w = ohT * alphaT
            readout = jnp.dot(w, hh,
                              preferred_element_type=jnp.float32)
            q_star = jnp.concatenate([q, readout], axis=1)
        z = jnp.maximum(
            jnp.dot(q_star, wc1[...],
                    preferred_element_type=jnp.float32) + bc1_[...], 0.0)
        o_ref[...] = jnp.dot(z, wc2[...],
                             preferred_element_type=jnp.float32) + bc2_[...]

    full = lambda shape: pl.BlockSpec(shape, lambda: tuple(0 for _ in shape))
    return pl.pallas_call(
        body,
        in_specs=[full((N_NODES, H)), full((1, N_NODES)),
                  full((2 * H, 4 * H)), full((H, 4 * H)),
                  full((1, 4 * H)), full((1, 4 * H)),
                  full((H, 4 * H)), full((H, 4 * H)),
                  full((1, 4 * H)), full((1, 4 * H)),
                  full((2 * H, H)), full((1, H)),
                  full((H, N_OUT)), full((1, N_OUT))],
        out_specs=full((N_GRAPHS, N_OUT)),
        out_shape=jax.ShapeDtypeStruct((N_GRAPHS, N_OUT), jnp.float32),
    )(h, gid2d, Wih0, Whh0, bih0, bhh0, Wih1, Whh1, bih1, bhh1,
      Wc1, bc1, Wc2, bc2)


def _sc_gather(table, idx2d):
    mesh = plsc.VectorSubcoreMesh(core_axis_name="c", subcore_axis_name="s")

    @functools.partial(
        pl.kernel, mesh=mesh, compiler_params=_SC_PARAMS,
        out_type=jax.ShapeDtypeStruct((E_PAD, H), jnp.float32),
        scratch_types=[pltpu.VMEM((ROWS_W, 128), jnp.int32),
                       pltpu.VMEM((2, SUP * 128, H), jnp.float32),
                       pltpu.VMEM_SHARED((N_PAD, H), jnp.float32),
                       pltpu.SemaphoreType.DMA],
    )
    def k(table_h, idx_h, out_h, idx_v, rows_v, table_s, sem):
        cid = lax.axis_index("c")
        sid = lax.axis_index("s")
        wid = sid * 2 + cid
        row0 = wid * ROWS_W

        pltpu.sync_copy(table_h.at[pl.ds(sid * RPS, RPS)],
                        table_s.at[pl.ds(sid * RPS, RPS)])
        plsc.subcore_barrier()

        pltpu.sync_copy(idx_h.at[pl.ds(row0, ROWS_W)], idx_v)
        ebase = row0 * 128

        def fire(s, b):
            return [pltpu.async_copy(table_s.at[idx_v.at[s * SUP + j]],
                                     rows_v.at[b].at[pl.ds(j * 128, 128)],
                                     sem)
                    for j in range(SUP)]

        ga = {0: fire(0, 0)}
        for s in range(NSUP):
            b = s % 2
            if s + 1 < NSUP:
                ga[(s + 1) % 2] = fire(s + 1, (s + 1) % 2)
            for hd in ga[b]:
                hd.wait()
            pltpu.sync_copy(rows_v.at[b],
                            out_h.at[pl.ds(ebase + s * SUP * 128,
                                           SUP * 128)])

    return k(table, idx2d).reshape(EROWS, PK * H)


def _sc_scatter_add(msg4, dst2d, zero_init):
    mesh = plsc.VectorSubcoreMesh(core_axis_name="c", subcore_axis_name="s")

    @functools.partial(
        pl.kernel, mesh=mesh, compiler_params=_SC_PARAMS,
        out_type=jax.ShapeDtypeStruct((2, N_PAD, H), jnp.float32),
        scratch_types=[pltpu.VMEM((ROWS_W, 128), jnp.int32),
                       pltpu.VMEM((2, SUP * 128, H), jnp.float32),
                       pltpu.VMEM_SHARED((N_PAD, H), jnp.float32),
                       pltpu.SemaphoreType.DMA,
                       pltpu.SemaphoreType.DMA],
    )
    def k(msg_h, idx_h, zro_h, out_h, idx_v, msg_v, shared, sem_ld, sem_sc):
        cid = lax.axis_index("c")
        sid = lax.axis_index("s")
        wid = sid * 2 + cid
        row0 = wid * ROWS_W
        ebase = row0 * 128

        def load(s, b):
            return pltpu.async_copy(
                msg_h.at[pl.ds(ebase + s * SUP * 128, SUP * 128)],
                msg_v.at[b], sem_ld)

        ld = {0: load(0, 0)}
        pltpu.sync_copy(idx_h.at[pl.ds(row0, ROWS_W)], idx_v)
        pltpu.sync_copy(zro_h.at[pl.ds(sid * RPS, RPS)],
                        shared.at[pl.ds(sid * RPS, RPS)])
        plsc.subcore_barrier()
        sc = {0: [], 1: []}
        for s in range(NSUP):
            b = s % 2
            nb = (s + 1) % 2
            if s + 1 < NSUP:
                for hd in sc[nb]:
                    hd.wait()
                sc[nb] = []
                ld[nb] = load(s + 1, nb)
            ld[b].wait()
            sc[b] = [pltpu.async_copy(msg_v.at[b].at[pl.ds(j * 128, 128)],
                                      shared.at[idx_v.at[s * SUP + j]],
                                      sem_sc, add=True)
                     for j in range(SUP)]
        for bb in (0, 1):
            for hd in sc[bb]:
                hd.wait()
        plsc.subcore_barrier()
        pltpu.sync_copy(shared.at[pl.ds(sid * RPS, RPS)],
                        out_h.at[cid, pl.ds(sid * RPS, RPS)])

    return k(msg4.reshape(E_PAD, H), dst2d, zero_init)


def _sc_degree(dst2d, zero_init, ones_rows):
    mesh = plsc.VectorSubcoreMesh(core_axis_name="c", subcore_axis_name="s")

    @functools.partial(
        pl.kernel, mesh=mesh, compiler_params=_SC_PARAMS,
        out_type=jax.ShapeDtypeStruct((2, N_PAD, H), jnp.float32),
        scratch_types=[pltpu.VMEM((ROWS_W, 128), jnp.int32),
                       pltpu.VMEM((128, H), jnp.float32),
                       pltpu.VMEM_SHARED((N_PAD, H), jnp.float32),
                       pltpu.SemaphoreType.DMA],
    )
    def k(idx_h, zro_h, ones_h, out_h, idx_v, ones_v, shared, sem):
        cid = lax.axis_index("c")
        sid = lax.axis_index("s")
        wid = sid * 2 + cid
        row0 = wid * ROWS_W
        pltpu.sync_copy(zro_h.at[pl.ds(sid * RPS, RPS)],
                        shared.at[pl.ds(sid * RPS, RPS)])
        plsc.subcore_barrier()
        pltpu.sync_copy(ones_h, ones_v)
        pltpu.sync_copy(idx_h.at[pl.ds(row0, ROWS_W)], idx_v)
        sc = []
        for r in range(ROWS_W):
            sc.append(pltpu.async_copy(ones_v, shared.at[idx_v.at[r]],
                                       sem, add=True))
        for hd in sc:
            hd.wait()
        plsc.subcore_barrier()
        pltpu.sync_copy(shared.at[pl.ds(sid * RPS, RPS)],
                        out_h.at[cid, pl.ds(sid * RPS, RPS)])

    return k(dst2d, zero_init, ones_rows)


def kernel(x_node, x_edge, edge_index, node_graph_ids,
           Wn, bn, We, be, Wen1, ben1, Wen2, ben2,
           gru_Wih, gru_Whh, gru_bih, gru_bhh,
           lstm_Wih0, lstm_Whh0, lstm_bih0, lstm_bhh0,
           lstm_Wih1, lstm_Whh1, lstm_bih1, lstm_bhh1,
           Wc1, bc1, Wc2, bc2):
    f32 = jnp.float32
    src = edge_index[0]
    dst = edge_index[1]
    pad = E_PAD - N_EDGES
    src2d = jnp.concatenate(
        [src, jnp.zeros((pad,), jnp.int32)]).reshape(E_PAD // 128, 128)
    dst2d = jnp.concatenate(
        [dst, jnp.full((pad,), N_NODES, jnp.int32)]).reshape(E_PAD // 128, 128)
    xe4 = x_edge.reshape(N_EDGES // PK, PK * N_BOND)
    xn4 = jnp.concatenate(
        [x_node.reshape(N_NODES // PK, PK * N_ATOM),
         jnp.zeros((NROWS - N_NODES // PK, PK * N_ATOM), f32)])
    eyePK = jnp.eye(PK, dtype=f32)
    W4n = jnp.kron(eyePK, Wn)
    b4n = jnp.tile(bn, PK).reshape(1, PK * H)
    W4e = jnp.kron(eyePK, We)
    W14 = jnp.kron(eyePK, Wen1)
    be4 = jnp.tile(be, PK).reshape(1, PK * H)
    b14 = jnp.tile(ben1, PK).reshape(1, PK * H)
    W2rT = Wen2.reshape(H, H, H).transpose(2, 0, 1).reshape(H, H * H)
    Ben2T4 = jnp.kron(eyePK, ben2.reshape(H, H).T)
    def gru_pack_w(W):
        return (jnp.kron(eyePK, W).reshape(PK * H, PK, 3, H)
                .transpose(0, 2, 1, 3).reshape(PK * H, 3 * PK * H))

    def gru_pack_b(b):
        return jnp.tile(b.reshape(3, H), (1, PK)).reshape(1, 3 * PK * H)
    Wihp = gru_pack_w(gru_Wih)
    Whhp = gru_pack_w(gru_Whh)
    bihp = gru_pack_b(gru_bih)
    bhhp = gru_pack_b(gru_bhh)
    zeroN = jnp.zeros((N_PAD, H), f32)
    ones_rows = jnp.ones((128, H), f32)
    gid2d = node_graph_ids.reshape(1, N_NODES)

    h4 = _node_embed(xn4, W4n, b4n)
    degp4 = _sc_degree(dst2d, zeroN, ones_rows).reshape(2, NROWS, PK * H)

    hsrc4 = _sc_gather(h4.reshape(N_PAD, H), src2d)
    msg4, t4 = _edge_messages_fused(xe4, hsrc4, W4e, be4, W14, b14,
                                    W2rT, Ben2T4)
    aggp4 = _sc_scatter_add(msg4, dst2d, zeroN).reshape(2, NROWS, PK * H)
    h4 = _gru_update(aggp4, degp4, h4, Wihp, Whhp, bihp, bhhp)
    hsrc4 = _sc_gather(h4.reshape(N_PAD, H), src2d)
    msg4 = _edge_messages(t4, hsrc4, W2rT, Ben2T4)
    aggp4 = _sc_scatter_add(msg4, dst2d, zeroN).reshape(2, NROWS, PK * H)
    h4 = _gru_update(aggp4, degp4, h4, Wihp, Whhp, bihp, bhhp)

    h_fin = h4.reshape(N_PAD, H)[:N_NODES]
    out = _set2set(h_fin, gid2d,
                   lstm_Wih0, lstm_Whh0, lstm_bih0.reshape(1, 4 * H),
                   lstm_bhh0.reshape(1, 4 * H),
                   lstm_Wih1, lstm_Whh1, lstm_bih1.reshape(1, 4 * H),
                   lstm_bhh1.reshape(1, 4 * H),
                   Wc1, bc1.reshape(1, H), Wc2, bc2.reshape(1, N_OUT))
    return out
